# baseline (speedup 1.0000x reference)
import functools
import math

import jax
import jax.numpy as jnp
from jax import lax
from jax.experimental import pallas as pl
from jax.experimental.pallas import tpu as pltpu
from jax.experimental.pallas import tpu_sc as plsc

N = 10000
DEG = 16
D = 256
TPERC = 0.45
REMOVE = math.floor(DEG * TPERC)

NC = 2
NS = 16
LB = 32
NW = NC * NS

CH = 16
ROWS = CH * DEG
NCHUNKS = N // CH
KMAX = -(-NCHUNKS // NW)
GRP = D // LB


def _mm_body(x_ref, wp_ref, o_ref):
    dn = (((1,), (1,)), ((), ()))
    y = lax.dot_general(x_ref[...], wp_ref[...], dn,
                        preferred_element_type=jnp.float32
                        ).astype(jnp.bfloat16)
    yu = lax.bitcast_convert_type(y, jnp.uint16).astype(jnp.uint32)
    pe, po = yu[:, : D // 2], yu[:, D // 2:]
    o_ref[...] = lax.bitcast_convert_type(pe | (po << 16), jnp.float32)


def _project(x, w):
    m = x.shape[0]
    blk = 2000
    grid = m // blk
    wp = jnp.concatenate([w[0::2], w[1::2]])
    return pl.pallas_call(
        _mm_body,
        grid=(grid,),
        in_specs=[
            pl.BlockSpec((blk, D), lambda i: (i, 0)),
            pl.BlockSpec((D, D), lambda i: (0, 0)),
        ],
        out_specs=pl.BlockSpec((blk, D // 2), lambda i: (i, 0)),
        out_shape=jax.ShapeDtypeStruct((m, D // 2), jnp.float32),
    )(x, wp)


_SORT8 = ((0, 1), (2, 3), (4, 5), (6, 7), (0, 2), (1, 3), (4, 6), (5, 7),
          (1, 2), (5, 6), (0, 4), (1, 5), (2, 6), (3, 7), (2, 4), (3, 5),
          (1, 2), (3, 4), (5, 6))


def _median16(vs):
    vs = list(vs)
    for off in (0, 8):
        for (i, j) in _SORT8:
            a, b = vs[off + i], vs[off + j]
            vs[off + i] = jnp.minimum(a, b)
            vs[off + j] = jnp.maximum(a, b)
    lo = [jnp.minimum(vs[i], vs[15 - i]) for i in range(8)]
    hi = [jnp.maximum(vs[i], vs[15 - i]) for i in range(8)]
    while len(lo) > 1:
        lo = [jnp.maximum(lo[k], lo[k + 1]) for k in range(0, len(lo), 2)]
        hi = [jnp.minimum(hi[k], hi[k + 1]) for k in range(0, len(hi), 2)]
    return (lo[0] + hi[0]) * 0.5


_sc_mesh = plsc.VectorSubcoreMesh(core_axis_name="c", subcore_axis_name="s")


@functools.partial(
    pl.kernel,
    mesh=_sc_mesh,
    out_type=jax.ShapeDtypeStruct((N * D,), jnp.float32),
    scratch_types=[
        pltpu.VMEM((ROWS,), jnp.int32),
        pltpu.VMEM((ROWS,), jnp.int32),
        pltpu.VMEM((ROWS, D // 2), jnp.float32),
        pltpu.VMEM((ROWS, D // 2), jnp.float32),
        pltpu.VMEM((CH * D,), jnp.float32),
        pltpu.VMEM((CH * D,), jnp.float32),
        pltpu.SemaphoreType.DMA,
        pltpu.SemaphoreType.DMA,
        pltpu.SemaphoreType.DMA,
        pltpu.SemaphoreType.DMA,
        pltpu.SemaphoreType.DMA,
        pltpu.SemaphoreType.DMA,
    ],
    compiler_params=pltpu.CompilerParams(
        use_tc_tiling_on_sc=False, needs_layout_passes=False
    ),
)
def _sc_median(h_hbm, nbrs_hbm, out_hbm, idx0, idx1, rows0, rows1,
               outv0, outv1, isem0, isem1, gsem0, gsem1, osem0, osem1):
    cid = lax.axis_index("c")
    sid = lax.axis_index("s")
    wid = cid * NS + sid
    idx_b = (idx0, idx1)
    rows_b = (rows0, rows1)
    out_b = (outv0, outv1)
    isem_b = (isem0, isem1)
    gsem_b = (gsem0, gsem1)
    osem_b = (osem0, osem1)
    iota = lax.iota(jnp.int32, 16)

    def _idx_copy(k, b):
        chunk = wid + NW * k
        return pltpu.make_async_copy(
            nbrs_hbm.at[pl.ds(chunk * ROWS, ROWS)], idx_b[b], isem_b[b])

    def _gather_copy(b):
        return pltpu.make_async_copy(h_hbm.at[idx_b[b]], rows_b[b], gsem_b[b])

    def _out_copy(k, b):
        chunk = wid + NW * k
        return pltpu.make_async_copy(
            out_b[b], out_hbm.at[pl.ds(chunk * CH * D, CH * D)], osem_b[b])

    def _live(k):
        return jnp.logical_and(k >= 0, wid + NW * k < NCHUNKS)

    def idx_start(k, b):
        pl.when(_live(k))(lambda: _idx_copy(k, b).start())

    def idx_wait(k, b):
        pl.when(_live(k))(lambda: _idx_copy(k, b).wait())

    def gather_start(k, b):
        pl.when(_live(k))(lambda: _gather_copy(b).start())

    def gather_wait(k, b):
        pl.when(_live(k))(lambda: _gather_copy(b).wait())

    def out_start(k, b):
        pl.when(_live(k))(lambda: _out_copy(k, b).start())

    def out_wait(k, b):
        pl.when(_live(k))(lambda: _out_copy(k, b).wait())

    def compute(k, b):
        rows_v = rows_b[b]
        out_v = out_b[b]

        @pl.when(_live(k))
        def _():
            @plsc.parallel_loop(0, CH * GRP, unroll=3)
            def grp_body(i):
                n = i // GRP
                gg = lax.rem(i, GRP)
                cs = pl.ds(gg * (LB // 2), LB // 2)
                vs = [
                    plsc.bitcast(rows_v[n * DEG + j, cs], jnp.bfloat16)
                    for j in range(DEG)
                ]
                med = _median16(vs)
                ev, od = plsc.unpack(med, format=plsc.PackFormat.INTERLEAVED,
                                     preferred_element_type=jnp.float32)
                ia = (n * D + gg * LB) + 2 * iota
                plsc.store_scatter(out_v, [ia], ev)
                plsc.store_scatter(out_v, [ia + 1], od)

    idx_start(0, 0)
    idx_start(1, 1)
    idx_wait(0, 0)
    gather_start(0, 0)

    def step(k, b):
        gather_wait(k, b)
        idx_start(k + 2, b)
        idx_wait(k + 1, 1 - b)
        gather_start(k + 1, 1 - b)
        out_wait(k - 2, b)
        compute(k, b)
        out_start(k, b)

    def pair_body(k2, carry):
        k = 2 * k2
        step(k, 0)
        step(k + 1, 1)
        return carry

    lax.fori_loop(0, KMAX // 2, pair_body, 0)
    out_wait(KMAX - 2, 0)
    out_wait(KMAX - 1, 1)


def kernel(x, nbrs, W):
    h = _project(x, W)
    return _sc_median(h, nbrs.reshape(-1)).reshape(N, D)


# speedup vs baseline: 1.0205x; 1.0205x over previous
import functools
import math

import jax
import jax.numpy as jnp
from jax import lax
from jax.experimental import pallas as pl
from jax.experimental.pallas import tpu as pltpu
from jax.experimental.pallas import tpu_sc as plsc

N = 10000
DEG = 16
D = 256
TPERC = 0.45
REMOVE = math.floor(DEG * TPERC)

NC = 2
NS = 16
LB = 32
NW = NC * NS

CH = 16
ROWS = CH * DEG
NCHUNKS = N // CH
KMAX = -(-NCHUNKS // NW)
GRP = D // LB


def _mm_body(x_ref, wp_ref, o_ref):
    dn = (((1,), (1,)), ((), ()))
    y = lax.dot_general(x_ref[...], wp_ref[...], dn,
                        preferred_element_type=jnp.float32
                        ).astype(jnp.bfloat16)
    yu = lax.bitcast_convert_type(y, jnp.uint16).astype(jnp.uint32)
    pe, po = yu[:, : D // 2], yu[:, D // 2:]
    o_ref[...] = lax.bitcast_convert_type(pe | (po << 16), jnp.float32)


def _project(x, w):
    m = x.shape[0]
    blk = 2000
    grid = m // blk
    wp = jnp.concatenate([w[0::2], w[1::2]])
    return pl.pallas_call(
        _mm_body,
        grid=(grid,),
        in_specs=[
            pl.BlockSpec((blk, D), lambda i: (i, 0)),
            pl.BlockSpec((D, D), lambda i: (0, 0)),
        ],
        out_specs=pl.BlockSpec((blk, D // 2), lambda i: (i, 0)),
        out_shape=jax.ShapeDtypeStruct((m, D // 2), jnp.float32),
    )(x, wp)


_SORT8 = ((0, 1), (2, 3), (4, 5), (6, 7), (0, 2), (1, 3), (4, 6), (5, 7),
          (1, 2), (5, 6), (0, 4), (1, 5), (2, 6), (3, 7), (2, 4), (3, 5),
          (1, 2), (3, 4), (5, 6))


def _median16(vs):
    vs = list(vs)
    for off in (0, 8):
        for (i, j) in _SORT8:
            a, b = vs[off + i], vs[off + j]
            vs[off + i] = jnp.minimum(a, b)
            vs[off + j] = jnp.maximum(a, b)
    lo = [jnp.minimum(vs[i], vs[15 - i]) for i in range(8)]
    hi = [jnp.maximum(vs[i], vs[15 - i]) for i in range(8)]
    while len(lo) > 1:
        lo = [jnp.maximum(lo[k], lo[k + 1]) for k in range(0, len(lo), 2)]
        hi = [jnp.minimum(hi[k], hi[k + 1]) for k in range(0, len(hi), 2)]
    return (lo[0] + hi[0]) * 0.5


_sc_mesh = plsc.VectorSubcoreMesh(core_axis_name="c", subcore_axis_name="s")


@functools.partial(
    pl.kernel,
    mesh=_sc_mesh,
    out_type=jax.ShapeDtypeStruct((N * D,), jnp.float32),
    scratch_types=[
        pltpu.VMEM((ROWS,), jnp.int32),
        pltpu.VMEM((ROWS,), jnp.int32),
        pltpu.VMEM((ROWS, D // 2), jnp.float32),
        pltpu.VMEM((ROWS, D // 2), jnp.float32),
        pltpu.VMEM((CH * D,), jnp.float32),
        pltpu.VMEM((CH * D,), jnp.float32),
        pltpu.SemaphoreType.DMA,
        pltpu.SemaphoreType.DMA,
        pltpu.SemaphoreType.DMA,
        pltpu.SemaphoreType.DMA,
        pltpu.SemaphoreType.DMA,
        pltpu.SemaphoreType.DMA,
    ],
    compiler_params=pltpu.CompilerParams(
        use_tc_tiling_on_sc=False, needs_layout_passes=False
    ),
)
def _sc_median(h_hbm, nbrs_hbm, out_hbm, idx0, idx1, rows0, rows1,
               outv0, outv1, isem0, isem1, gsem0, gsem1, osem0, osem1):
    cid = lax.axis_index("c")
    sid = lax.axis_index("s")
    wid = cid * NS + sid
    idx_b = (idx0, idx1)
    rows_b = (rows0, rows1)
    out_b = (outv0, outv1)
    isem_b = (isem0, isem1)
    gsem_b = (gsem0, gsem1)
    osem_b = (osem0, osem1)
    iota = lax.iota(jnp.int32, 16)

    def _idx_copy(k, b):
        chunk = wid + NW * k
        return pltpu.make_async_copy(
            nbrs_hbm.at[pl.ds(chunk * ROWS, ROWS)], idx_b[b], isem_b[b])

    def _gather_copy(b):
        return pltpu.make_async_copy(h_hbm.at[idx_b[b]], rows_b[b], gsem_b[b])

    def _out_copy(k, b):
        chunk = wid + NW * k
        return pltpu.make_async_copy(
            out_b[b], out_hbm.at[pl.ds(chunk * CH * D, CH * D)], osem_b[b])

    def _live(k):
        return jnp.logical_and(k >= 0, wid + NW * k < NCHUNKS)

    def idx_start(k, b):
        pl.when(_live(k))(lambda: _idx_copy(k, b).start())

    def idx_wait(k, b):
        pl.when(_live(k))(lambda: _idx_copy(k, b).wait())

    def gather_start(k, b):
        pl.when(_live(k))(lambda: _gather_copy(b).start())

    def gather_wait(k, b):
        pl.when(_live(k))(lambda: _gather_copy(b).wait())

    def out_start(k, b):
        pl.when(_live(k))(lambda: _out_copy(k, b).start())

    def out_wait(k, b):
        pl.when(_live(k))(lambda: _out_copy(k, b).wait())

    def compute(k, b):
        rows_v = rows_b[b]
        out_v = out_b[b]

        @pl.when(_live(k))
        def _():
            @plsc.parallel_loop(0, CH * GRP, unroll=2)
            def grp_body(i):
                n = i // GRP
                gg = lax.rem(i, GRP)
                cs = pl.ds(gg * (LB // 2), LB // 2)
                vs = [
                    plsc.bitcast(rows_v[n * DEG + j, cs], jnp.bfloat16)
                    for j in range(DEG)
                ]
                med = _median16(vs)
                ev, od = plsc.unpack(med, format=plsc.PackFormat.INTERLEAVED,
                                     preferred_element_type=jnp.float32)
                ia = (n * D + gg * LB) + 2 * iota
                plsc.store_scatter(out_v, [ia], ev)
                plsc.store_scatter(out_v, [ia + 1], od)

    idx_start(0, 0)
    idx_start(1, 1)
    idx_wait(0, 0)
    gather_start(0, 0)

    def step(k, b):
        gather_wait(k, b)
        idx_start(k + 2, b)
        idx_wait(k + 1, 1 - b)
        gather_start(k + 1, 1 - b)
        out_wait(k - 2, b)
        compute(k, b)
        out_start(k, b)

    def pair_body(k2, carry):
        k = 2 * k2
        step(k, 0)
        step(k + 1, 1)
        return carry

    lax.fori_loop(0, KMAX // 2, pair_body, 0)
    out_wait(KMAX - 2, 0)
    out_wait(KMAX - 1, 1)


def kernel(x, nbrs, W):
    h = _project(x, W)
    return _sc_median(h, nbrs.reshape(-1)).reshape(N, D)
